# baseline (speedup 1.0000x reference)
import jax
import jax.numpy as jnp
from jax import lax
from jax.experimental import pallas as pl
from jax.experimental.pallas import tpu as pltpu
from jax.experimental.pallas import tpu_sc as plsc

_NC = 2
_NS = 16
_NW = _NC * _NS
_L = 16

_ROWS = 16384
_COLS = 200
_N = _ROWS * _COLS
_K = 8
_UNIT = _K * 8 * 128
_UNITS = _N // _UNIT
_UPW_LO = _UNITS // _NW
_EXTRA = _UNITS - _UPW_LO * _NW
_SLOTS = _UPW_LO + 1
_UPJ = 128 // _K
_JROW = 2 * _ROWS
_KC = _K * 256
_VPJ = _K * 8


def _sc_body(ids_hbm, w_hbm, out_hbm,
             w20_v, table0_v, table1_v, ids_v0, ids_v1, out_v0, out_v1,
             in_s0, in_s1, out_s0, out_s1):
    wid = lax.axis_index("s") * _NC + lax.axis_index("c")
    n_units = _UPW_LO + jnp.where(wid < _EXTRA, 1, 0)
    u_base = wid * _UPW_LO + jnp.minimum(wid, _EXTRA)
    pltpu.sync_copy(w_hbm, w20_v)
    iota = lax.iota(jnp.int32, _L)
    in_tab = iota < 10
    iota2 = jnp.where(in_tab, iota * 2, 0)
    table0_v[...] = plsc.load_gather(w20_v, [iota2], mask=in_tab)
    table1_v[...] = plsc.load_gather(w20_v, [iota2 + 1], mask=in_tab)
    ids_bufs = (ids_v0, ids_v1)
    out_bufs = (out_v0, out_v1)
    in_sems = (in_s0, in_s1)
    out_sems = (out_s0, out_s1)

    def issue_in(u, buf, sem):
        pltpu.async_copy(ids_hbm.at[pl.ds((u_base + u) * _UNIT, _UNIT)],
                         buf, sem)

    def drain(buf, sem):
        pltpu.make_async_copy(ids_hbm.at[pl.ds(0, _UNIT)], buf, sem).wait() \
            if buf.shape == (_UNIT,) else \
            pltpu.make_async_copy(out_hbm.at[pl.ds(0, 2 * _UNIT)], buf,
                                  sem).wait()

    issue_in(0, ids_bufs[0], in_sems[0])

    def do_unit(u, b):
        u_g = u_base + u
        jt = u_g // _UPJ
        itg = u_g % _UPJ
        q0 = (jt * 8) * _JROW + itg * _KC
        ids_v = ids_bufs[b]
        out_v = out_bufs[b]

        @pl.when(u + 1 < n_units)
        def _():
            issue_in(u + 1, ids_bufs[1 - b], in_sems[1 - b])

        drain(ids_v, in_sems[b])

        @pl.when(u >= 2)
        def _():
            drain(out_v, out_sems[b])

        def js_body(js, carry):
            @plsc.parallel_loop(0, _VPJ, unroll=8)
            def vec_body(q):
                it_r = q >> 3
                lb = (q & 7) * _L
                boff = it_r * 1024 + js * 128 + lb
                oo = js * _KC + it_r * 256 + lb
                idv = ids_v[pl.ds(boff, _L)]
                v0 = plsc.load_gather(table0_v, [idv])
                v1 = plsc.load_gather(table1_v, [idv])
                out_v[pl.ds(oo, _L)] = v0
                out_v[pl.ds(oo + 128, _L)] = v1
            pltpu.async_copy(out_v.at[pl.ds(js * _KC, _KC)],
                             out_hbm.at[pl.ds(q0 + js * _JROW, _KC)],
                             out_sems[b])
            return carry

        lax.fori_loop(0, 8, js_body, 0)

    def pair_body(ko, carry):
        for b in (0, 1):
            u = ko * 2 + b

            @pl.when(u < n_units)
            def _():
                do_unit(u, b)
        return carry

    lax.fori_loop(0, (_SLOTS + 1) // 2, pair_body, 0)
    drain(out_bufs[0], out_sems[0])
    drain(out_bufs[1], out_sems[1])


@jax.jit
def kernel(input_ids, weight):
    ids_lin = (input_ids.astype(jnp.int32)
               .reshape(128, 128, 25, 8)
               .transpose(2, 0, 3, 1)
               .reshape(-1))
    w_lin = weight.reshape(-1)
    mesh = plsc.VectorSubcoreMesh(
        core_axis_name="c", subcore_axis_name="s",
        num_cores=_NC, num_subcores=_NS)
    out_lin = pl.kernel(
        _sc_body,
        out_type=jax.ShapeDtypeStruct((2 * _N,), jnp.float32),
        mesh=mesh,
        scratch_types=[
            pltpu.VMEM((20,), jnp.float32),
            pltpu.VMEM((_L,), jnp.float32),
            pltpu.VMEM((_L,), jnp.float32),
            pltpu.VMEM((_UNIT,), jnp.int32),
            pltpu.VMEM((_UNIT,), jnp.int32),
            pltpu.VMEM((2 * _UNIT,), jnp.float32),
            pltpu.VMEM((2 * _UNIT,), jnp.float32),
            pltpu.SemaphoreType.DMA,
            pltpu.SemaphoreType.DMA,
            pltpu.SemaphoreType.DMA,
            pltpu.SemaphoreType.DMA,
        ],
        compiler_params=pltpu.CompilerParams(needs_layout_passes=False),
    )(ids_lin, w_lin)
    return (out_lin.reshape(_COLS, 128, 2, 128)
            .transpose(1, 3, 0, 2)
            .reshape(_ROWS, _COLS, 2))
